# baseline (speedup 1.0000x reference)
import dataclasses
import functools

import jax
import jax.numpy as jnp
from jax import lax
from jax.experimental import pallas as pl
from jax.experimental.pallas import tpu as pltpu
from jax.experimental.pallas import tpu_sc as plsc

B, L, V, D = 4096, 200, 100000, 128
NC, NS = 2, 16
NW = NC * NS
BPW = B // NW
LANES = 16
DW = D // 2
NCH = DW // LANES
C1 = 128
C2 = L - C1


def _bag(elements, sizes, packed):
    mesh = plsc.VectorSubcoreMesh(core_axis_name="c", subcore_axis_name="s")
    cp = pltpu.CompilerParams(use_tc_tiling_on_sc=False)
    if "needs_layout_passes" in pltpu.CompilerParams.__dataclass_fields__:
        cp = dataclasses.replace(cp, needs_layout_passes=False)

    @functools.partial(
        pl.kernel,
        compiler_params=cp,
        out_type=jax.ShapeDtypeStruct((B, D), jnp.float32),
        mesh=mesh,
        scratch_types=[
            pltpu.VMEM((BPW, L), jnp.int32),
            pltpu.VMEM((2, L, DW), jnp.int32),
            pltpu.VMEM((BPW, D), jnp.float32),
            pltpu.VMEM((BPW,), jnp.float32),
            pltpu.SemaphoreType.DMA,
            pltpu.SemaphoreType.DMA,
        ],
    )
    def k(elements_hbm, sizes_hbm, packed_hbm, out_hbm,
          idx_v, rows_v, out_v, sizes_s, sem0, sem1):
        wid = lax.axis_index("s") * NC + lax.axis_index("c")
        base = wid * BPW
        pltpu.sync_copy(elements_hbm.at[pl.ds(base, BPW)], idx_v)
        pltpu.sync_copy(sizes_hbm.at[pl.ds(base, BPW)], sizes_s)
        sems = (sem0, sem1)

        def gather_ops(i, buf):
            sem = sems[buf]
            return (
                pltpu.make_async_copy(
                    packed_hbm.at[idx_v.at[i, pl.ds(0, C1)]],
                    rows_v.at[buf, pl.ds(0, C1)], sem),
                pltpu.make_async_copy(
                    packed_hbm.at[idx_v.at[i, pl.ds(C1, C2)]],
                    rows_v.at[buf, pl.ds(C1, C2)], sem),
            )

        def issue(i, buf):
            for cp in gather_ops(i, buf):
                cp.start()

        def wait(buf):
            for cp in gather_ops(0, buf):
                cp.wait()

        issue(0, 0)

        @pl.loop(0, BPW // LANES)
        def _(gi):
            inv = 1.0 / sizes_s[pl.ds(gi * LANES, LANES)]
            for j in range(LANES):
                i = gi * LANES + j
                cur = j % 2
                nxt = 1 - cur

                @pl.when(i < BPW - 1)
                def _():
                    issue(i + 1, nxt)

                wait(cur)

                def body(jj, acc):
                    new = []
                    for c in range(NCH):
                        w = rows_v[cur, jj, pl.ds(c * LANES, LANES)]
                        bf = plsc.bitcast(w, jnp.bfloat16)
                        a, b = plsc.unpack(
                            bf, format=plsc.PackFormat.INTERLEAVED)
                        new.append(acc[2 * c] + a)
                        new.append(acc[2 * c + 1] + b)
                    return tuple(new)

                acc = lax.fori_loop(
                    0, L, body,
                    tuple(jnp.zeros((LANES,), jnp.float32)
                          for _ in range(2 * NCH)),
                    unroll=2)

                rows_ids = jnp.full((LANES,), i, jnp.int32)
                evens = 2 * lax.iota(jnp.int32, LANES)
                for c in range(NCH):
                    cols = 32 * c + evens
                    plsc.store_scatter(
                        out_v, [rows_ids, cols], acc[2 * c] * inv[j])
                    plsc.store_scatter(
                        out_v, [rows_ids, cols + 1], acc[2 * c + 1] * inv[j])

        pltpu.sync_copy(out_v, out_hbm.at[pl.ds(base, BPW)])

    return k(elements, sizes, packed)


def kernel(elements, sizes, table):
    packed = lax.bitcast_convert_type(
        table.astype(jnp.bfloat16).reshape(V, DW, 2), jnp.int32)
    return _bag(elements.astype(jnp.int32), sizes, packed)


# speedup vs baseline: 2.3910x; 2.3910x over previous
import dataclasses
import functools

import jax
import jax.numpy as jnp
from jax import lax
from jax.experimental import pallas as pl
from jax.experimental.pallas import tpu as pltpu
from jax.experimental.pallas import tpu_sc as plsc

B, L, V, D = 4096, 200, 100000, 128
NC, NS = 2, 16
NW = NC * NS
BPW = B // NW
LANES = 16
NCH = D // (2 * LANES)
C1 = 128
C2 = L - C1


def _bag(elements_flat, sizes, tbf):
    mesh = plsc.VectorSubcoreMesh(core_axis_name="c", subcore_axis_name="s")
    cp = pltpu.CompilerParams(use_tc_tiling_on_sc=False)
    if "needs_layout_passes" in pltpu.CompilerParams.__dataclass_fields__:
        cp = dataclasses.replace(cp, needs_layout_passes=False)

    @functools.partial(
        pl.kernel,
        compiler_params=cp,
        out_type=jax.ShapeDtypeStruct((B * D,), jnp.float32),
        mesh=mesh,
        scratch_types=[
            pltpu.VMEM((BPW * L,), jnp.int32),
            pltpu.VMEM((2, L, D), jnp.bfloat16),
            pltpu.VMEM((BPW * D,), jnp.float32),
            pltpu.VMEM((BPW,), jnp.float32),
            pltpu.SemaphoreType.DMA,
            pltpu.SemaphoreType.DMA,
        ],
    )
    def k(elements_hbm, sizes_hbm, tbf_hbm, out_hbm,
          idx_v, rows_v, out_v, sizes_s, sem0, sem1):
        wid = lax.axis_index("s") * NC + lax.axis_index("c")
        base = wid * BPW
        pltpu.sync_copy(elements_hbm.at[pl.ds(base * L, BPW * L)], idx_v)
        pltpu.sync_copy(sizes_hbm.at[pl.ds(base, BPW)], sizes_s)
        sems = (sem0, sem1)

        def gather_ops(i, buf):
            sem = sems[buf]
            return (
                pltpu.make_async_copy(
                    tbf_hbm.at[idx_v.at[pl.ds(i * L, C1)]],
                    rows_v.at[buf, pl.ds(0, C1)], sem),
                pltpu.make_async_copy(
                    tbf_hbm.at[idx_v.at[pl.ds(i * L + C1, C2)]],
                    rows_v.at[buf, pl.ds(C1, C2)], sem),
            )

        def issue(i, buf):
            for cp_ in gather_ops(i, buf):
                cp_.start()

        def wait(buf):
            for cp_ in gather_ops(0, buf):
                cp_.wait()

        issue(0, 0)

        @pl.loop(0, BPW // LANES)
        def _(gi):
            inv = 1.0 / sizes_s[pl.ds(gi * LANES, LANES)]
            for j in range(LANES):
                i = gi * LANES + j
                cur = j % 2
                nxt = 1 - cur

                @pl.when(i < BPW - 1)
                def _():
                    issue(i + 1, nxt)

                wait(cur)

                def body(jj, acc):
                    new = []
                    for c in range(NCH):
                        bf = rows_v[cur, jj, pl.ds(c * 2 * LANES, 2 * LANES)]
                        a, b = plsc.unpack(
                            bf, format=plsc.PackFormat.INTERLEAVED)
                        new.append(acc[2 * c] + a)
                        new.append(acc[2 * c + 1] + b)
                    return tuple(new)

                acc = lax.fori_loop(
                    0, L, body,
                    tuple(jnp.zeros((LANES,), jnp.float32)
                          for _ in range(2 * NCH)),
                    unroll=2)

                obase = i * D
                evens = 2 * lax.iota(jnp.int32, LANES)
                for c in range(NCH):
                    cols = obase + 32 * c + evens
                    plsc.store_scatter(
                        out_v, [cols], acc[2 * c] * inv[j])
                    plsc.store_scatter(
                        out_v, [cols + 1], acc[2 * c + 1] * inv[j])

        pltpu.sync_copy(out_v, out_hbm.at[pl.ds(base * D, BPW * D)])

    return k(elements_flat, sizes, tbf)


def kernel(elements, sizes, table):
    out = _bag(elements.astype(jnp.int32).reshape(-1), sizes,
               table.astype(jnp.bfloat16))
    return out.reshape(B, D)


# speedup vs baseline: 2.4592x; 1.0285x over previous
import dataclasses
import functools

import jax
import jax.numpy as jnp
from jax import lax
from jax.experimental import pallas as pl
from jax.experimental.pallas import tpu as pltpu
from jax.experimental.pallas import tpu_sc as plsc

B, L, V, D = 4096, 200, 100000, 128
NC, NS = 2, 16
NW = NC * NS
BPW = B // NW
LANES = 16
DW = D // 2
NCH = DW // LANES
C1 = 128
C2 = L - C1
VPW = V // NW
CR = 125
NCHUNK = VPW // CR


def _sc_params():
    cp = pltpu.CompilerParams(use_tc_tiling_on_sc=False)
    if "needs_layout_passes" in pltpu.CompilerParams.__dataclass_fields__:
        cp = dataclasses.replace(cp, needs_layout_passes=False)
    return cp


def _mesh():
    return plsc.VectorSubcoreMesh(core_axis_name="c", subcore_axis_name="s")


def _pack(table):
    @functools.partial(
        pl.kernel,
        compiler_params=_sc_params(),
        out_type=jax.ShapeDtypeStruct((V, DW), jnp.int32),
        mesh=_mesh(),
        scratch_types=[
            pltpu.VMEM((CR, D), jnp.float32),
            pltpu.VMEM((CR, DW), jnp.int32),
            pltpu.SemaphoreType.DMA,
        ],
    )
    def k(table_hbm, out_hbm, in_v, out_v, sem):
        wid = lax.axis_index("s") * NC + lax.axis_index("c")
        base = wid * VPW
        iota = lax.iota(jnp.int32, LANES)

        @pl.loop(0, NCHUNK)
        def _(g):
            row0 = base + g * CR
            pltpu.async_copy(
                table_hbm.at[pl.ds(row0, CR)], in_v, sem).wait()

            def body(j, _):
                rows = jnp.full((LANES,), j, jnp.int32)
                for c in range(NCH):
                    ev = plsc.load_gather(in_v, [rows, 32 * c + 2 * iota])
                    od = plsc.load_gather(in_v, [rows, 32 * c + 2 * iota + 1])
                    bf = plsc.pack(ev, od, format=plsc.PackFormat.INTERLEAVED)
                    out_v[j, pl.ds(c * LANES, LANES)] = plsc.bitcast(
                        bf, jnp.int32)
                return 0

            lax.fori_loop(0, CR, body, 0, unroll=4)
            pltpu.async_copy(out_v, out_hbm.at[pl.ds(row0, CR)], sem).wait()

    return k(table)


def _bag(elements_flat, sizes, packed):
    @functools.partial(
        pl.kernel,
        compiler_params=_sc_params(),
        out_type=jax.ShapeDtypeStruct((B * D,), jnp.float32),
        mesh=_mesh(),
        scratch_types=[
            pltpu.VMEM((BPW * L,), jnp.int32),
            pltpu.VMEM((2, L, DW), jnp.int32),
            pltpu.VMEM((BPW * D,), jnp.float32),
            pltpu.VMEM((BPW,), jnp.float32),
            pltpu.SemaphoreType.DMA,
            pltpu.SemaphoreType.DMA,
        ],
    )
    def k(elements_hbm, sizes_hbm, packed_hbm, out_hbm,
          idx_v, rows_v, out_v, sizes_s, sem0, sem1):
        wid = lax.axis_index("s") * NC + lax.axis_index("c")
        base = wid * BPW
        pltpu.sync_copy(elements_hbm.at[pl.ds(base * L, BPW * L)], idx_v)
        pltpu.sync_copy(sizes_hbm.at[pl.ds(base, BPW)], sizes_s)
        sems = (sem0, sem1)

        def gather_ops(i, buf):
            sem = sems[buf]
            return (
                pltpu.make_async_copy(
                    packed_hbm.at[idx_v.at[pl.ds(i * L, C1)]],
                    rows_v.at[buf, pl.ds(0, C1)], sem),
                pltpu.make_async_copy(
                    packed_hbm.at[idx_v.at[pl.ds(i * L + C1, C2)]],
                    rows_v.at[buf, pl.ds(C1, C2)], sem),
            )

        def issue(i, buf):
            for cp_ in gather_ops(i, buf):
                cp_.start()

        def wait(buf):
            for cp_ in gather_ops(0, buf):
                cp_.wait()

        issue(0, 0)

        @pl.loop(0, BPW // LANES)
        def _(gi):
            inv = 1.0 / sizes_s[pl.ds(gi * LANES, LANES)]
            for j in range(LANES):
                i = gi * LANES + j
                cur = j % 2
                nxt = 1 - cur

                @pl.when(i < BPW - 1)
                def _():
                    issue(i + 1, nxt)

                wait(cur)

                def body(jj, acc):
                    new = []
                    for c in range(NCH):
                        w = rows_v[cur, jj, pl.ds(c * LANES, LANES)]
                        bf = plsc.bitcast(w, jnp.bfloat16)
                        a, b = plsc.unpack(
                            bf, format=plsc.PackFormat.INTERLEAVED)
                        new.append(acc[2 * c] + a)
                        new.append(acc[2 * c + 1] + b)
                    return tuple(new)

                acc = lax.fori_loop(
                    0, L, body,
                    tuple(jnp.zeros((LANES,), jnp.float32)
                          for _ in range(2 * NCH)),
                    unroll=2)

                obase = i * D
                evens = 2 * lax.iota(jnp.int32, LANES)
                for c in range(NCH):
                    cols = obase + 32 * c + evens
                    plsc.store_scatter(
                        out_v, [cols], acc[2 * c] * inv[j])
                    plsc.store_scatter(
                        out_v, [cols + 1], acc[2 * c + 1] * inv[j])

        pltpu.sync_copy(out_v, out_hbm.at[pl.ds(base * D, BPW * D)])

    return k(elements_flat, sizes, packed)


def kernel(elements, sizes, table):
    packed = _pack(table)
    out = _bag(elements.astype(jnp.int32).reshape(-1), sizes, packed)
    return out.reshape(B, D)


# speedup vs baseline: 2.9173x; 1.1863x over previous
import dataclasses
import functools

import jax
import jax.numpy as jnp
from jax import lax
from jax.experimental import pallas as pl
from jax.experimental.pallas import tpu as pltpu
from jax.experimental.pallas import tpu_sc as plsc

B, L, V, D = 4096, 200, 100000, 128
NC, NS = 2, 16
NW = NC * NS
BPW = B // NW
LANES = 16
DW = D // 2
NCH = DW // LANES
C1 = 128
C2 = L - C1
VPW = V // NW
CR = 125
NCHUNK = VPW // CR


def _sc_params():
    cp = pltpu.CompilerParams(use_tc_tiling_on_sc=False)
    if "needs_layout_passes" in pltpu.CompilerParams.__dataclass_fields__:
        cp = dataclasses.replace(cp, needs_layout_passes=False)
    return cp


def _mesh():
    return plsc.VectorSubcoreMesh(core_axis_name="c", subcore_axis_name="s")


def _pack(table):
    @functools.partial(
        pl.kernel,
        compiler_params=_sc_params(),
        out_type=jax.ShapeDtypeStruct((V, DW), jnp.int32),
        mesh=_mesh(),
        scratch_types=[
            pltpu.VMEM((2, CR, D), jnp.float32),
            pltpu.VMEM((2, CR, DW), jnp.int32),
            pltpu.SemaphoreType.DMA,
            pltpu.SemaphoreType.DMA,
            pltpu.SemaphoreType.DMA,
            pltpu.SemaphoreType.DMA,
        ],
    )
    def k(table_hbm, out_hbm, in_v, out_v, si0, si1, so0, so1):
        wid = lax.axis_index("s") * NC + lax.axis_index("c")
        base = wid * VPW
        sin = (si0, si1)
        sout = (so0, so1)

        def in_op(g, buf):
            return pltpu.make_async_copy(
                table_hbm.at[pl.ds(base + g * CR, CR)],
                in_v.at[buf], sin[buf])

        def out_op(g, buf):
            return pltpu.make_async_copy(
                out_v.at[buf],
                out_hbm.at[pl.ds(base + g * CR, CR)], sout[buf])

        in_op(0, 0).start()
        for g in range(NCHUNK):
            buf = g % 2
            if g + 1 < NCHUNK:
                in_op(g + 1, 1 - buf).start()
            in_op(g, buf).wait()
            if g >= 2:
                out_op(g - 2, buf).wait()

            def body(j, _):
                for c in range(NCH):
                    a = in_v[buf, j, pl.ds(32 * c, LANES)]
                    b = in_v[buf, j, pl.ds(32 * c + LANES, LANES)]
                    bf = plsc.pack(a, b, format=plsc.PackFormat.INTERLEAVED)
                    out_v[buf, j, pl.ds(c * LANES, LANES)] = plsc.bitcast(
                        bf, jnp.int32)
                return 0

            lax.fori_loop(0, CR, body, 0, unroll=5)
            out_op(g, buf).start()
        out_op(NCHUNK - 2, NCHUNK % 2).wait()
        out_op(NCHUNK - 1, (NCHUNK - 1) % 2).wait()

    return k(table)


def _bag(elements_flat, sizes, packed):
    @functools.partial(
        pl.kernel,
        compiler_params=_sc_params(),
        out_type=jax.ShapeDtypeStruct((B * D,), jnp.float32),
        mesh=_mesh(),
        scratch_types=[
            pltpu.VMEM((BPW * L,), jnp.int32),
            pltpu.VMEM((2, L, DW), jnp.int32),
            pltpu.VMEM((BPW * D,), jnp.float32),
            pltpu.VMEM((BPW,), jnp.float32),
            pltpu.SemaphoreType.DMA,
            pltpu.SemaphoreType.DMA,
        ],
    )
    def k(elements_hbm, sizes_hbm, packed_hbm, out_hbm,
          idx_v, rows_v, out_v, sizes_s, sem0, sem1):
        wid = lax.axis_index("s") * NC + lax.axis_index("c")
        base = wid * BPW
        pltpu.sync_copy(elements_hbm.at[pl.ds(base * L, BPW * L)], idx_v)
        pltpu.sync_copy(sizes_hbm.at[pl.ds(base, BPW)], sizes_s)
        sems = (sem0, sem1)

        def gather_ops(i, buf):
            sem = sems[buf]
            return (
                pltpu.make_async_copy(
                    packed_hbm.at[idx_v.at[pl.ds(i * L, C1)]],
                    rows_v.at[buf, pl.ds(0, C1)], sem),
                pltpu.make_async_copy(
                    packed_hbm.at[idx_v.at[pl.ds(i * L + C1, C2)]],
                    rows_v.at[buf, pl.ds(C1, C2)], sem),
            )

        def issue(i, buf):
            for cp_ in gather_ops(i, buf):
                cp_.start()

        def wait(buf):
            for cp_ in gather_ops(0, buf):
                cp_.wait()

        issue(0, 0)

        @pl.loop(0, BPW // LANES)
        def _(gi):
            inv = 1.0 / sizes_s[pl.ds(gi * LANES, LANES)]
            for j in range(LANES):
                i = gi * LANES + j
                cur = j % 2
                nxt = 1 - cur

                @pl.when(i < BPW - 1)
                def _():
                    issue(i + 1, nxt)

                wait(cur)

                def body(jj, acc):
                    new = []
                    for c in range(NCH):
                        w = rows_v[cur, jj, pl.ds(c * LANES, LANES)]
                        bf = plsc.bitcast(w, jnp.bfloat16)
                        a, b = plsc.unpack(
                            bf, format=plsc.PackFormat.INTERLEAVED)
                        new.append(acc[2 * c] + a)
                        new.append(acc[2 * c + 1] + b)
                    return tuple(new)

                acc = lax.fori_loop(
                    0, L, body,
                    tuple(jnp.zeros((LANES,), jnp.float32)
                          for _ in range(2 * NCH)),
                    unroll=2)

                obase = i * D
                for c in range(NCH):
                    out_v[pl.ds(obase + 32 * c, LANES)] = (
                        acc[2 * c] * inv[j])
                    out_v[pl.ds(obase + 32 * c + LANES, LANES)] = (
                        acc[2 * c + 1] * inv[j])

        pltpu.sync_copy(out_v, out_hbm.at[pl.ds(base * D, BPW * D)])

    return k(elements_flat, sizes, packed)


def kernel(elements, sizes, table):
    packed = _pack(table)
    out = _bag(elements.astype(jnp.int32).reshape(-1), sizes, packed)
    return out.reshape(B, D)


# speedup vs baseline: 3.0413x; 1.0425x over previous
import dataclasses
import functools

import jax
import jax.numpy as jnp
from jax import lax
from jax.experimental import pallas as pl
from jax.experimental.pallas import tpu as pltpu
from jax.experimental.pallas import tpu_sc as plsc

B, L, V, D = 4096, 200, 100000, 128
NC, NS = 2, 16
NW = NC * NS
BPW = B // NW
LANES = 16
DW = D // 2
NCH = DW // LANES
C1 = 128
C2 = L - C1
VPW = V // NW
CR = 125
NCHUNK = VPW // CR


def _sc_params():
    cp = pltpu.CompilerParams(use_tc_tiling_on_sc=False)
    if "needs_layout_passes" in pltpu.CompilerParams.__dataclass_fields__:
        cp = dataclasses.replace(cp, needs_layout_passes=False)
    return cp


def _mesh():
    return plsc.VectorSubcoreMesh(core_axis_name="c", subcore_axis_name="s")


def _pack(table):
    @functools.partial(
        pl.kernel,
        compiler_params=_sc_params(),
        out_type=jax.ShapeDtypeStruct((V, DW), jnp.int32),
        mesh=_mesh(),
        scratch_types=[
            pltpu.VMEM((2, CR, D), jnp.float32),
            pltpu.VMEM((2, CR, DW), jnp.int32),
            pltpu.SemaphoreType.DMA,
            pltpu.SemaphoreType.DMA,
            pltpu.SemaphoreType.DMA,
            pltpu.SemaphoreType.DMA,
        ],
    )
    def k(table_hbm, out_hbm, in_v, out_v, si0, si1, so0, so1):
        wid = lax.axis_index("s") * NC + lax.axis_index("c")
        base = wid * VPW
        sin = (si0, si1)
        sout = (so0, so1)

        def in_op(g, buf):
            return pltpu.make_async_copy(
                table_hbm.at[pl.ds(base + g * CR, CR)],
                in_v.at[buf], sin[buf])

        def out_op(g, buf):
            return pltpu.make_async_copy(
                out_v.at[buf],
                out_hbm.at[pl.ds(base + g * CR, CR)], sout[buf])

        in_op(0, 0).start()
        for g in range(NCHUNK):
            buf = g % 2
            if g + 1 < NCHUNK:
                in_op(g + 1, 1 - buf).start()
            in_op(g, buf).wait()
            if g >= 2:
                out_op(g - 2, buf).wait()

            def body(j, _):
                for c in range(NCH):
                    a = in_v[buf, j, pl.ds(32 * c, LANES)]
                    b = in_v[buf, j, pl.ds(32 * c + LANES, LANES)]
                    bf = plsc.pack(a, b, format=plsc.PackFormat.INTERLEAVED)
                    out_v[buf, j, pl.ds(c * LANES, LANES)] = plsc.bitcast(
                        bf, jnp.int32)
                return 0

            lax.fori_loop(0, CR, body, 0, unroll=5)
            out_op(g, buf).start()
        out_op(NCHUNK - 2, NCHUNK % 2).wait()
        out_op(NCHUNK - 1, (NCHUNK - 1) % 2).wait()

    return k(table)


def _bag(elements_flat, sizes, packed):
    @functools.partial(
        pl.kernel,
        compiler_params=_sc_params(),
        out_type=jax.ShapeDtypeStruct((B * D,), jnp.float32),
        mesh=_mesh(),
        scratch_types=[
            pltpu.VMEM((BPW * L,), jnp.int32),
            pltpu.VMEM((2, L, DW), jnp.int32),
            pltpu.VMEM((BPW * D,), jnp.float32),
            pltpu.VMEM((BPW,), jnp.float32),
            pltpu.SemaphoreType.DMA,
            pltpu.SemaphoreType.DMA,
        ],
    )
    def k(elements_hbm, sizes_hbm, packed_hbm, out_hbm,
          idx_v, rows_v, out_v, sizes_s, sem0, sem1):
        wid = lax.axis_index("s") * NC + lax.axis_index("c")
        base = wid * BPW
        pltpu.sync_copy(elements_hbm.at[pl.ds(base * L, BPW * L)], idx_v)
        pltpu.sync_copy(sizes_hbm.at[pl.ds(base, BPW)], sizes_s)
        sems = (sem0, sem1)

        def gather_ops(i, buf):
            sem = sems[buf]
            return (
                pltpu.make_async_copy(
                    packed_hbm.at[idx_v.at[pl.ds(i * L, C1)]],
                    rows_v.at[buf, pl.ds(0, C1)], sem),
                pltpu.make_async_copy(
                    packed_hbm.at[idx_v.at[pl.ds(i * L + C1, C2)]],
                    rows_v.at[buf, pl.ds(C1, C2)], sem),
            )

        def issue(i, buf):
            for cp_ in gather_ops(i, buf):
                cp_.start()

        def wait(buf):
            for cp_ in gather_ops(0, buf):
                cp_.wait()

        issue(0, 0)

        @pl.loop(0, BPW // LANES)
        def _(gi):
            inv = 1.0 / sizes_s[pl.ds(gi * LANES, LANES)]
            for j in range(LANES):
                i = gi * LANES + j
                cur = j % 2
                nxt = 1 - cur

                @pl.when(i < BPW - 1)
                def _():
                    issue(i + 1, nxt)

                wait(cur)

                def body(jj, acc):
                    new = []
                    for c in range(NCH):
                        w0 = rows_v[cur, 2 * jj, pl.ds(c * LANES, LANES)]
                        w1 = rows_v[cur, 2 * jj + 1, pl.ds(c * LANES, LANES)]
                        bf = (plsc.bitcast(w0, jnp.bfloat16)
                              + plsc.bitcast(w1, jnp.bfloat16))
                        a, b = plsc.unpack(
                            bf, format=plsc.PackFormat.INTERLEAVED)
                        new.append(acc[2 * c] + a)
                        new.append(acc[2 * c + 1] + b)
                    return tuple(new)

                acc = lax.fori_loop(
                    0, L // 2, body,
                    tuple(jnp.zeros((LANES,), jnp.float32)
                          for _ in range(2 * NCH)),
                    unroll=2)

                obase = i * D
                for c in range(NCH):
                    out_v[pl.ds(obase + 32 * c, LANES)] = (
                        acc[2 * c] * inv[j])
                    out_v[pl.ds(obase + 32 * c + LANES, LANES)] = (
                        acc[2 * c + 1] * inv[j])

        pltpu.sync_copy(out_v, out_hbm.at[pl.ds(base * D, BPW * D)])

    return k(elements_flat, sizes, packed)


def kernel(elements, sizes, table):
    packed = _pack(table)
    out = _bag(elements.astype(jnp.int32).reshape(-1), sizes, packed)
    return out.reshape(B, D)
